# speedup vs baseline: 1.5703x; 1.5703x over previous
import functools

import jax
import jax.numpy as jnp
from jax import lax
from jax.experimental import pallas as pl
from jax.experimental.pallas import tpu as pltpu
from jax.experimental.pallas import tpu_sc as plsc

VOCAB = 1000000
EMBED = 128
BATCH = 4096

_NC = 2
_NS = 16
_NW = _NC * _NS
_BPW = BATCH // _NW

_mesh = plsc.VectorSubcoreMesh(
    core_axis_name="c", subcore_axis_name="s", num_cores=_NC, num_subcores=_NS
)


@functools.partial(
    pl.kernel,
    out_type=(
        jax.ShapeDtypeStruct((BATCH, EMBED), jnp.float32),
        jax.ShapeDtypeStruct((BATCH, EMBED), jnp.float32),
    ),
    mesh=_mesh,
    scratch_types=[
        pltpu.VMEM((_BPW,), jnp.int32),
        pltpu.VMEM((_BPW,), jnp.int32),
        pltpu.VMEM((_BPW, EMBED), jnp.float32),
        pltpu.VMEM((_BPW, EMBED), jnp.float32),
        pltpu.SemaphoreType.DMA,
        pltpu.SemaphoreType.DMA,
    ],
)
def _sc_gather(target_hbm, context_hbm, in_tab_hbm, out_tab_hbm,
               in_emb_hbm, out_emb_hbm,
               tgt_idx_v, ctx_idx_v, in_rows_v, out_rows_v, sem_a, sem_b):
    wid = lax.axis_index("s") * _NC + lax.axis_index("c")
    base = wid * _BPW
    pltpu.sync_copy(target_hbm.at[pl.ds(base, _BPW)], tgt_idx_v)
    pltpu.sync_copy(context_hbm.at[pl.ds(base, _BPW)], ctx_idx_v)
    ga = pltpu.async_copy(in_tab_hbm.at[tgt_idx_v], in_rows_v, sem_a)
    gb = pltpu.async_copy(out_tab_hbm.at[ctx_idx_v], out_rows_v, sem_b)
    ga.wait()
    wa = pltpu.async_copy(in_rows_v, in_emb_hbm.at[pl.ds(base, _BPW)], sem_a)
    gb.wait()
    wb = pltpu.async_copy(out_rows_v, out_emb_hbm.at[pl.ds(base, _BPW)], sem_b)
    wa.wait()
    wb.wait()


_BM = 512
_BN = 512


def _mm_body(a_ref, b_ref, o_ref):
    o_ref[...] = lax.dot_general(
        a_ref[...], b_ref[...],
        dimension_numbers=(((1,), (1,)), ((), ())),
        preferred_element_type=jnp.float32,
    )


_matmul = pl.pallas_call(
    _mm_body,
    grid=(BATCH // _BM, BATCH // _BN),
    in_specs=[
        pl.BlockSpec((_BM, EMBED), lambda i, j: (i, 0)),
        pl.BlockSpec((_BN, EMBED), lambda i, j: (j, 0)),
    ],
    out_specs=pl.BlockSpec((_BM, _BN), lambda i, j: (i, j)),
    out_shape=jax.ShapeDtypeStruct((BATCH, BATCH), jnp.float32),
)


def kernel(target, context, in_table, out_table):
    target = target.astype(jnp.int32)
    context = context.astype(jnp.int32)
    in_embeds, out_embeds = _sc_gather(target, context, in_table, out_table)
    return _matmul(in_embeds, out_embeds)


# speedup vs baseline: 2.7078x; 1.7244x over previous
import functools

import jax
import jax.numpy as jnp
from jax import lax
from jax.experimental import pallas as pl
from jax.experimental.pallas import tpu as pltpu
from jax.experimental.pallas import tpu_sc as plsc

VOCAB = 1000000
EMBED = 128
BATCH = 4096

_NC = 2
_NS = 16
_NW = _NC * _NS
_BPW = BATCH // _NW

_mesh = plsc.VectorSubcoreMesh(
    core_axis_name="c", subcore_axis_name="s", num_cores=_NC, num_subcores=_NS
)


@functools.partial(
    pl.kernel,
    out_type=(
        jax.ShapeDtypeStruct((BATCH, EMBED), jnp.float32),
        jax.ShapeDtypeStruct((BATCH, EMBED), jnp.float32),
    ),
    mesh=_mesh,
    scratch_types=[
        pltpu.VMEM((_BPW,), jnp.int32),
        pltpu.VMEM((_BPW,), jnp.int32),
        pltpu.VMEM((_BPW, EMBED), jnp.float32),
        pltpu.VMEM((_BPW, EMBED), jnp.float32),
        pltpu.SemaphoreType.DMA,
        pltpu.SemaphoreType.DMA,
    ],
)
def _sc_gather(target_hbm, context_hbm, in_tab_hbm, out_tab_hbm,
               in_emb_hbm, out_emb_hbm,
               tgt_idx_v, ctx_idx_v, in_rows_v, out_rows_v, sem_a, sem_b):
    wid = lax.axis_index("s") * _NC + lax.axis_index("c")
    base = wid * _BPW
    pltpu.sync_copy(target_hbm.at[pl.ds(base, _BPW)], tgt_idx_v)
    pltpu.sync_copy(context_hbm.at[pl.ds(base, _BPW)], ctx_idx_v)
    ga = pltpu.async_copy(in_tab_hbm.at[tgt_idx_v], in_rows_v, sem_a)
    gb = pltpu.async_copy(out_tab_hbm.at[ctx_idx_v], out_rows_v, sem_b)
    ga.wait()
    wa = pltpu.async_copy(in_rows_v, in_emb_hbm.at[pl.ds(base, _BPW)], sem_a)
    gb.wait()
    wb = pltpu.async_copy(out_rows_v, out_emb_hbm.at[pl.ds(base, _BPW)], sem_b)
    wa.wait()
    wb.wait()


_BM = 512


def _mm_body(a_ref, b_ref, o_ref):
    o_ref[...] = lax.dot_general(
        a_ref[...], b_ref[...],
        dimension_numbers=(((1,), (1,)), ((), ())),
        preferred_element_type=jnp.float32,
    )


_matmul = pl.pallas_call(
    _mm_body,
    grid=(BATCH // _BM,),
    in_specs=[
        pl.BlockSpec((_BM, EMBED), lambda i: (i, 0)),
        pl.BlockSpec((BATCH, EMBED), lambda i: (0, 0)),
    ],
    out_specs=pl.BlockSpec((_BM, BATCH), lambda i: (i, 0)),
    out_shape=jax.ShapeDtypeStruct((BATCH, BATCH), jnp.float32),
)


def kernel(target, context, in_table, out_table):
    target = target.astype(jnp.int32)
    context = context.astype(jnp.int32)
    in_embeds, out_embeds = _sc_gather(target, context, in_table, out_table)
    return _matmul(in_embeds, out_embeds)
